# Initial kernel scaffold; baseline (speedup 1.0000x reference)
import jax
import jax.numpy as jnp
from jax.experimental import pallas as pl


def kernel(cell_indices, weight):
    raise NotImplementedError("write your pallas kernel here")


# speedup vs baseline: 1.8764x; 1.8764x over previous
import functools

import jax
import jax.numpy as jnp
from jax import lax
from jax.experimental import pallas as pl
from jax.experimental.pallas import tpu as pltpu
from jax.experimental.pallas import tpu_sc as plsc

DIM = 64
BATCH = 16384
HIST = 50
TOTAL = BATCH * HIST
NW = 32
B_PER_W = TOTAL // NW
SLICE = 128
N_SL = B_PER_W // SLICE
NBUF = 8
N_RND = N_SL // NBUF

_mesh = plsc.VectorSubcoreMesh(core_axis_name="c", subcore_axis_name="s")


@functools.partial(
    pl.kernel,
    mesh=_mesh,
    out_type=jax.ShapeDtypeStruct((TOTAL, DIM), jnp.float32),
    scratch_types=[
        pltpu.VMEM((N_SL, SLICE), jnp.int32),
        pltpu.VMEM((NBUF, SLICE, DIM), jnp.float32),
    ] + [pltpu.SemaphoreType.DMA] * (2 * NBUF),
    compiler_params=pltpu.CompilerParams(use_tc_tiling_on_sc=False),
)
def _gather_all(idx_hbm, table_hbm, out_hbm, idx_v, rows_v, *sems):
    gsem = sems[:NBUF]
    wsem = sems[NBUF:]
    wid = lax.axis_index("s") * 2 + lax.axis_index("c")
    base_sl = wid * N_SL
    base_row = wid * B_PER_W

    pltpu.sync_copy(idx_hbm.at[pl.ds(base_sl, N_SL)], idx_v)

    for b in range(NBUF):
        pltpu.async_copy(table_hbm.at[idx_v.at[b]], rows_v.at[b], gsem[b])

    def rnd(g, carry):
        for b in range(NBUF):
            c = g * NBUF + b
            pltpu.make_async_copy(table_hbm.at[pl.ds(0, SLICE)],
                                  rows_v.at[b], gsem[b]).wait()
            pltpu.async_copy(rows_v.at[b],
                             out_hbm.at[pl.ds(base_row + c * SLICE, SLICE)],
                             wsem[b])

            @pl.when(g < N_RND - 1)
            def _():
                pltpu.make_async_copy(rows_v.at[b],
                                      out_hbm.at[pl.ds(0, SLICE)],
                                      wsem[b]).wait()
                pltpu.async_copy(table_hbm.at[idx_v.at[c + NBUF]],
                                 rows_v.at[b], gsem[b])

        return carry

    lax.fori_loop(0, N_RND, rnd, 0)

    for b in range(NBUF):
        pltpu.make_async_copy(rows_v.at[b], out_hbm.at[pl.ds(0, SLICE)],
                              wsem[b]).wait()


def kernel(cell_indices, weight):
    idx = cell_indices.astype(jnp.int32).reshape(TOTAL // SLICE, SLICE)
    out = _gather_all(idx, weight)
    return out.reshape(BATCH, HIST, DIM)


# speedup vs baseline: 1.8780x; 1.0008x over previous
import functools

import jax
import jax.numpy as jnp
from jax import lax
from jax.experimental import pallas as pl
from jax.experimental.pallas import tpu as pltpu
from jax.experimental.pallas import tpu_sc as plsc

DIM = 64
BATCH = 16384
HIST = 50
NW = 32
B_PER_W = BATCH // NW
SLICE = 128
KB = B_PER_W // SLICE
N_UNITS = HIST * KB
NBUF = 4
N_RND = N_UNITS // NBUF

_mesh = plsc.VectorSubcoreMesh(core_axis_name="c", subcore_axis_name="s")


@functools.partial(
    pl.kernel,
    mesh=_mesh,
    out_type=jax.ShapeDtypeStruct((BATCH, HIST, DIM), jnp.float32),
    scratch_types=[
        pltpu.VMEM((HIST, B_PER_W), jnp.int32),
        pltpu.VMEM((NBUF, SLICE, DIM), jnp.float32),
    ] + [pltpu.SemaphoreType.DMA] * (2 * NBUF),
    compiler_params=pltpu.CompilerParams(use_tc_tiling_on_sc=False),
)
def _gather_all(idx_hbm, table_hbm, out_hbm, idx_v, rows_v, *sems):
    gsem = sems[:NBUF]
    wsem = sems[NBUF:]
    wid = lax.axis_index("s") * 2 + lax.axis_index("c")
    base_b = wid * B_PER_W

    pltpu.sync_copy(idx_hbm.at[:, pl.ds(base_b, B_PER_W)], idx_v)

    def fire_gather(u, p):
        h = u // KB
        k = lax.rem(u, KB)
        pltpu.async_copy(table_hbm.at[idx_v.at[h, pl.ds(k * SLICE, SLICE)]],
                         rows_v.at[p], gsem[p])

    def fire_write(u, p):
        h = u // KB
        k = lax.rem(u, KB)
        pltpu.async_copy(rows_v.at[p],
                         out_hbm.at[pl.ds(base_b + k * SLICE, SLICE), h],
                         wsem[p])

    for p in range(NBUF):
        fire_gather(p, p)

    def rnd(g, carry):
        for p in range(NBUF):
            u = g * NBUF + p
            pltpu.make_async_copy(table_hbm.at[pl.ds(0, SLICE)],
                                  rows_v.at[p], gsem[p]).wait()
            fire_write(u, p)

            @pl.when(g < N_RND - 1)
            def _():
                pltpu.make_async_copy(rows_v.at[p],
                                      out_hbm.at[pl.ds(base_b, SLICE), 0],
                                      wsem[p]).wait()
                fire_gather(u + NBUF, p)

        return carry

    lax.fori_loop(0, N_RND, rnd, 0)

    for p in range(NBUF):
        pltpu.make_async_copy(rows_v.at[p],
                              out_hbm.at[pl.ds(base_b, SLICE), 0],
                              wsem[p]).wait()


def kernel(cell_indices, weight):
    return _gather_all(cell_indices.astype(jnp.int32).T, weight)
